# baseline (speedup 1.0000x reference)
import functools
import math

import jax
import jax.numpy as jnp
from jax import lax
from jax.experimental import pallas as pl
from jax.experimental.pallas import tpu as pltpu
from jax.experimental.pallas import tpu_sc as plsc

_B = 16
_N = 32768
_D = 1024
_TEMPERATURE = 1.0

_L = 16
_NTILES = 16
_CHUNK = _N // _NTILES
_NVEC = _CHUNK // _L
_DVEC = _D // _L

_NEG_INF = float("-inf")
_HALF_LOG_2PIE = 0.5 * math.log(2.0 * math.pi * math.e)
_LN2 = math.log(2.0)
_SQRT2 = math.sqrt(2.0)


def _vlog(x):
    bits = lax.bitcast_convert_type(x, jnp.int32)
    e = ((bits >> 23) & 0xFF) - 127
    m = lax.bitcast_convert_type(
        (bits & 0x7FFFFF) | 0x3F800000, jnp.float32
    )
    big = m > jnp.float32(_SQRT2)
    m = jnp.where(big, m * jnp.float32(0.5), m)
    e = e + jnp.where(big, jnp.int32(1), jnp.int32(0))
    t = (m - jnp.float32(1.0)) / (m + jnp.float32(1.0))
    t2 = t * t
    p = jnp.float32(2.0) * t * (
        jnp.float32(1.0)
        + t2 * (jnp.float32(1.0 / 3.0)
                + t2 * (jnp.float32(1.0 / 5.0) + t2 * jnp.float32(1.0 / 7.0)))
    )
    return e.astype(jnp.float32) * jnp.float32(_LN2) + p


def _sc_body(coh_hbm, batch_hbm, std_hbm, prio_hbm, norm_hbm, unc_hbm,
             coh_v, idx_v, s_v, e_v, n_v, std_v,
             u_v, gmax_v, sum_v, ginv_v, row_v, all_v,
             shared_u, shared_max, shared_sum):
    sid = lax.axis_index("s")
    base = sid * _CHUNK

    pltpu.sync_copy(std_hbm.at[pl.ds(sid * _D, _D)], std_v)
    pltpu.sync_copy(coh_hbm.at[pl.ds(base, _CHUNK)], coh_v)
    pltpu.sync_copy(batch_hbm.at[pl.ds(base, _CHUNK)], idx_v)

    lane = lax.iota(jnp.int32, _L)
    neg_inf_vec = jnp.full((_L,), _NEG_INF, dtype=jnp.float32)
    zero_vec = jnp.zeros((_L,), dtype=jnp.float32)
    inv_temp = jnp.float32(1.0 / _TEMPERATURE)

    def body_u(j, acc):
        sd = std_v[pl.ds(j * _L, _L)]
        return acc + _vlog(sd)

    ent = lax.fori_loop(0, _DVEC, body_u, zero_vec)
    u_t = jnp.sum(ent) + jnp.float32(_HALF_LOG_2PIE * _D)
    row_v[...] = jnp.where(lane == sid, u_t, zero_vec)
    pltpu.sync_copy(row_v, shared_u.at[pl.ds(sid * _L, _L)])
    plsc.subcore_barrier()
    pltpu.sync_copy(shared_u, all_v)
    u = zero_vec
    for t in range(_NTILES):
        u = u + all_v[pl.ds(t * _L, _L)]
    u_v[...] = u

    def body_a(j, accs):
        off = j * _L
        c = coh_v[pl.ds(off, _L)]
        ii = idx_v[pl.ds(off, _L)]
        ue = plsc.load_gather(u_v, [ii])
        s = (c * ue) * inv_temp
        s_v[pl.ds(off, _L)] = s
        return tuple(
            jnp.maximum(accs[b], jnp.where(ii == b, s, neg_inf_vec))
            for b in range(_B)
        )

    accs = lax.fori_loop(0, _NVEC, body_a, (neg_inf_vec,) * _B)

    lmax = neg_inf_vec
    for b in range(_B):
        lmax = jnp.where(lane == b, jnp.max(accs[b]), lmax)
    row_v[...] = lmax
    pltpu.sync_copy(row_v, shared_max.at[pl.ds(sid * _L, _L)])
    plsc.subcore_barrier()

    pltpu.sync_copy(shared_max, all_v)
    g = neg_inf_vec
    for t in range(_NTILES):
        g = jnp.maximum(g, all_v[pl.ds(t * _L, _L)])
    gmax_v[...] = g

    sum_v[...] = zero_vec

    def body_b(j, carry):
        off = j * _L
        s = s_v[pl.ds(off, _L)]
        ii = idx_v[pl.ds(off, _L)]
        gm = plsc.load_gather(gmax_v, [ii])
        e = jnp.exp(s - gm)
        e_v[pl.ds(off, _L)] = e
        plsc.addupdate_scatter(sum_v, [ii], e)
        return carry

    lax.fori_loop(0, _NVEC, body_b, jnp.int32(0))

    row_v[...] = sum_v[...]
    pltpu.sync_copy(row_v, shared_sum.at[pl.ds(sid * _L, _L)])
    plsc.subcore_barrier()

    pltpu.sync_copy(shared_sum, all_v)
    gs = zero_vec
    for t in range(_NTILES):
        gs = gs + all_v[pl.ds(t * _L, _L)]
    ginv_v[...] = jnp.float32(1.0) / gs

    def body_c(j, carry):
        off = j * _L
        e = e_v[pl.ds(off, _L)]
        ii = idx_v[pl.ds(off, _L)]
        iv = plsc.load_gather(ginv_v, [ii])
        n_v[pl.ds(off, _L)] = e * iv
        return carry

    lax.fori_loop(0, _NVEC, body_c, jnp.int32(0))

    pltpu.sync_copy(s_v, prio_hbm.at[pl.ds(base, _CHUNK)])
    pltpu.sync_copy(n_v, norm_hbm.at[pl.ds(base, _CHUNK)])

    @pl.when(sid == 0)
    def _():
        pltpu.sync_copy(u_v, unc_hbm)


def _sc_priority(coherence_spatial, batch, std_flat):
    mesh = plsc.VectorSubcoreMesh(
        core_axis_name="c", subcore_axis_name="s", num_cores=1
    )
    f32 = jnp.float32
    run = functools.partial(
        pl.kernel,
        mesh=mesh,
        out_type=[
            jax.ShapeDtypeStruct((_N,), f32),
            jax.ShapeDtypeStruct((_N,), f32),
            jax.ShapeDtypeStruct((_B,), f32),
        ],
        scratch_types=[
            pltpu.VMEM((_CHUNK,), f32),
            pltpu.VMEM((_CHUNK,), jnp.int32),
            pltpu.VMEM((_CHUNK,), f32),
            pltpu.VMEM((_CHUNK,), f32),
            pltpu.VMEM((_CHUNK,), f32),
            pltpu.VMEM((_D,), f32),
            pltpu.VMEM((_L,), f32),
            pltpu.VMEM((_L,), f32),
            pltpu.VMEM((_L,), f32),
            pltpu.VMEM((_L,), f32),
            pltpu.VMEM((_L,), f32),
            pltpu.VMEM((_NTILES * _L,), f32),
            pltpu.VMEM_SHARED((_NTILES * _L,), f32),
            pltpu.VMEM_SHARED((_NTILES * _L,), f32),
            pltpu.VMEM_SHARED((_NTILES * _L,), f32),
        ],
        compiler_params=pltpu.CompilerParams(needs_layout_passes=False),
    )(_sc_body)
    return run(coherence_spatial, batch, std_flat)


def kernel(coherence_spatial, posterior_mean, posterior_std, batch):
    priority, priority_normalized, uncertainty = _sc_priority(
        coherence_spatial, batch, posterior_std.reshape(-1)
    )
    return (priority, priority_normalized, uncertainty)


# speedup vs baseline: 1.0251x; 1.0251x over previous
import functools
import math

import jax
import jax.numpy as jnp
from jax import lax
from jax.experimental import pallas as pl
from jax.experimental.pallas import tpu as pltpu
from jax.experimental.pallas import tpu_sc as plsc

_B = 16
_N = 32768
_D = 1024
_TEMPERATURE = 1.0

_L = 16
_NTILES = 16
_CHUNK = _N // _NTILES
_NVEC = _CHUNK // _L

_NEG_INF = float("-inf")


def _uncertainty_body(std_ref, out_ref):
    s = std_ref[...]
    ent = 0.5 * jnp.log((2.0 * math.pi * math.e) * jnp.square(s))
    out_ref[...] = jnp.sum(ent, axis=1, keepdims=True)


def _tc_uncertainty(posterior_std):
    out = pl.pallas_call(
        _uncertainty_body,
        out_shape=jax.ShapeDtypeStruct((_B, 1), jnp.float32),
    )(posterior_std)
    return out.reshape(_B)


def _sc_body(coh_hbm, batch_hbm, u_hbm, prio_hbm, norm_hbm,
             coh_v, idx_v, s_v, e_v, n_v,
             u_v, lmax_v, fac_v, row_v, all_v,
             shared_rows, sem_in, sem_out):
    sid = lax.axis_index("s")
    base = sid * _CHUNK

    cp_coh = pltpu.make_async_copy(coh_hbm.at[pl.ds(base, _CHUNK)], coh_v, sem_in)
    cp_idx = pltpu.make_async_copy(batch_hbm.at[pl.ds(base, _CHUNK)], idx_v, sem_in)
    cp_u = pltpu.make_async_copy(u_hbm, u_v, sem_in)
    cp_coh.start()
    cp_idx.start()
    cp_u.start()
    cp_coh.wait()
    cp_idx.wait()
    cp_u.wait()

    lane = lax.iota(jnp.int32, _L)
    neg_inf_vec = jnp.full((_L,), _NEG_INF, dtype=jnp.float32)
    zero_vec = jnp.zeros((_L,), dtype=jnp.float32)
    inv_temp = jnp.float32(1.0 / _TEMPERATURE)

    def body_a(j, accs):
        off = j * _L
        c = coh_v[pl.ds(off, _L)]
        ii = idx_v[pl.ds(off, _L)]
        ue = plsc.load_gather(u_v, [ii])
        s = (c * ue) * inv_temp
        s_v[pl.ds(off, _L)] = s
        return tuple(
            jnp.maximum(accs[b], jnp.where(ii == b, s, neg_inf_vec))
            for b in range(_B)
        )

    accs = lax.fori_loop(0, _NVEC, body_a, (neg_inf_vec,) * _B, unroll=2)

    cp_prio = pltpu.make_async_copy(s_v, prio_hbm.at[pl.ds(base, _CHUNK)], sem_out)
    cp_prio.start()

    lmax = neg_inf_vec
    for b in range(_B):
        lmax = jnp.where(lane == b, jnp.max(accs[b]), lmax)
    lmax_v[...] = lmax

    fac_v[...] = zero_vec

    def body_b(j, carry):
        off = j * _L
        s = s_v[pl.ds(off, _L)]
        ii = idx_v[pl.ds(off, _L)]
        lm = plsc.load_gather(lmax_v, [ii])
        e = jnp.exp(s - lm)
        e_v[pl.ds(off, _L)] = e
        plsc.addupdate_scatter(fac_v, [ii], e)
        return carry

    lax.fori_loop(0, _NVEC, body_b, jnp.int32(0), unroll=2)

    row_v[pl.ds(0, _L)] = lmax_v[...]
    row_v[pl.ds(_L, _L)] = fac_v[...]
    pltpu.sync_copy(row_v, shared_rows.at[pl.ds(sid * (2 * _L), 2 * _L)])
    plsc.subcore_barrier()
    pltpu.sync_copy(shared_rows, all_v)

    g = neg_inf_vec
    for t in range(_NTILES):
        g = jnp.maximum(g, all_v[pl.ds(t * 2 * _L, _L)])
    total = zero_vec
    for t in range(_NTILES):
        lm_t = all_v[pl.ds(t * 2 * _L, _L)]
        ls_t = all_v[pl.ds(t * 2 * _L + _L, _L)]
        total = total + ls_t * jnp.exp(lm_t - g)
    fac_v[...] = jnp.exp(lmax_v[...] - g) / total

    def body_c(j, carry):
        off = j * _L
        e = e_v[pl.ds(off, _L)]
        ii = idx_v[pl.ds(off, _L)]
        fv = plsc.load_gather(fac_v, [ii])
        n_v[pl.ds(off, _L)] = e * fv
        return carry

    lax.fori_loop(0, _NVEC, body_c, jnp.int32(0), unroll=2)

    pltpu.sync_copy(n_v, norm_hbm.at[pl.ds(base, _CHUNK)])
    cp_prio.wait()


def _sc_softmax(coherence_spatial, batch, uncertainty):
    mesh = plsc.VectorSubcoreMesh(
        core_axis_name="c", subcore_axis_name="s", num_cores=1
    )
    f32 = jnp.float32
    run = functools.partial(
        pl.kernel,
        mesh=mesh,
        out_type=[
            jax.ShapeDtypeStruct((_N,), f32),
            jax.ShapeDtypeStruct((_N,), f32),
        ],
        scratch_types=[
            pltpu.VMEM((_CHUNK,), f32),
            pltpu.VMEM((_CHUNK,), jnp.int32),
            pltpu.VMEM((_CHUNK,), f32),
            pltpu.VMEM((_CHUNK,), f32),
            pltpu.VMEM((_CHUNK,), f32),
            pltpu.VMEM((_L,), f32),
            pltpu.VMEM((_L,), f32),
            pltpu.VMEM((_L,), f32),
            pltpu.VMEM((2 * _L,), f32),
            pltpu.VMEM((_NTILES * 2 * _L,), f32),
            pltpu.VMEM_SHARED((_NTILES * 2 * _L,), f32),
            pltpu.SemaphoreType.DMA,
            pltpu.SemaphoreType.DMA,
        ],
        compiler_params=pltpu.CompilerParams(needs_layout_passes=False),
    )(_sc_body)
    return run(coherence_spatial, batch, uncertainty)


def kernel(coherence_spatial, posterior_mean, posterior_std, batch):
    uncertainty = _tc_uncertainty(posterior_std)
    priority, priority_normalized = _sc_softmax(
        coherence_spatial, batch, uncertainty
    )
    return (priority, priority_normalized, uncertainty)
